# Initial kernel scaffold; baseline (speedup 1.0000x reference)
import jax
import jax.numpy as jnp
from jax.experimental import pallas as pl


def kernel(images, features, conv_w, conv_b, cls_w, cls_b, bbox_w, bbox_b):
    raise NotImplementedError("write your pallas kernel here")


# speedup vs baseline: 37.1333x; 37.1333x over previous
import numpy as np
import jax
import jax.numpy as jnp
from jax import lax
from jax.experimental import pallas as pl
from jax.experimental.pallas import tpu as pltpu

_GRID = 50
_STRIDE = 16
_A = 9
_PRE = 1000
_POST = 1000
_NMS_THRESH = 0.7
_MIN_SIZE = 1e-3
_BBOX_CLIP = float(np.log(1000.0 / 16.0))

_TH_HI = float(np.float32(_NMS_THRESH))
_TH_LO = float(np.float32(np.float64(_NMS_THRESH) - np.float64(np.float32(_NMS_THRESH))))
_EPS_HI = float(np.float32(1e-12))
_EPS_LO = float(np.float32(np.float64(1e-12) - np.float64(np.float32(1e-12))))

_HW = _GRID + 2
_ROWS = _HW * _HW
_RPAD = 2816


def _mk_anchors():
    scales = jnp.array([128.0, 256.0, 512.0], dtype=jnp.float32)
    ratios = jnp.array([0.5, 1.0, 2.0], dtype=jnp.float32)
    h_r = jnp.sqrt(ratios)
    w_r = 1.0 / h_r
    ws = (w_r[:, None] * scales[None, :]).reshape(-1)
    hs = (h_r[:, None] * scales[None, :]).reshape(-1)
    base = jnp.round(jnp.stack([-ws, -hs, ws, hs], axis=1) / 2.0)
    s = jnp.arange(_GRID, dtype=jnp.float32) * _STRIDE
    gy, gx = jnp.meshgrid(s, s, indexing='ij')
    gx = gx.reshape(-1)
    gy = gy.reshape(-1)
    shifts = jnp.stack([gx, gy, gx, gy], axis=1)
    return (shifts[:, None, :] + base[None, :, :]).reshape(-1, 4)


def _decode_batched(deltas, anchors):
    widths = anchors[:, 2] - anchors[:, 0]
    heights = anchors[:, 3] - anchors[:, 1]
    ctr_x = anchors[:, 0] + 0.5 * widths
    ctr_y = anchors[:, 1] + 0.5 * heights
    dx = deltas[..., 0]
    dy = deltas[..., 1]
    dw = jnp.minimum(deltas[..., 2], _BBOX_CLIP)
    dh = jnp.minimum(deltas[..., 3], _BBOX_CLIP)
    pcx = dx * widths + ctr_x
    pcy = dy * heights + ctr_y
    pw = jnp.exp(dw) * widths
    ph = jnp.exp(dh) * heights
    return jnp.stack(
        [pcx - 0.5 * pw, pcy - 0.5 * ph, pcx + 0.5 * pw, pcy + 0.5 * ph],
        axis=-1)


def _two_sum(a, b):
    s = a + b
    v = s - a
    return s, (a - (s - v)) + (b - v)


def _two_diff(a, b):
    s = a - b
    v = s - a
    return s, (a - (s - v)) - (b + v)


def _quick_two_sum(a, b):
    s = a + b
    return s, b - (s - a)


def _split32(a):
    t = jnp.float32(4097.0) * a
    hi = t - (t - a)
    return hi, a - hi


def _two_prod(a, b):
    p = a * b
    ah, al = _split32(a)
    bh, bl = _split32(b)
    return p, ((ah * bh - p) + ah * bl + al * bh) + al * bl


def _df_add(x, y):
    s, e = _two_sum(x[0], y[0])
    return _quick_two_sum(s, e + (x[1] + y[1]))


def _df_neg(x):
    return (-x[0], -x[1])


def _df_mul(x, y):
    p, e = _two_prod(x[0], y[0])
    return _quick_two_sum(p, e + (x[0] * y[1] + x[1] * y[0]))


def _df_gt(x, y):
    d = _df_add(x, _df_neg(y))
    return d[0] > 0


def _head_kernel(x_ref, wm_ref, cb_ref, wc_ref, cbb_ref, wb_ref, bbb_ref,
                 obj_ref, dl_ref):
    x = x_ref[0]
    acc = jnp.zeros((_ROWS, 256), jnp.float32)
    for k in range(9):
        dy, dx = divmod(k, 3)
        off = 53 + (dy - 1) * _HW + (dx - 1)
        xs = x[off:off + _ROWS, :]
        acc = acc + jnp.dot(xs, wm_ref[k], preferred_element_type=jnp.float32)
    t = jnp.maximum(acc + cb_ref[...], 0.0)
    obj_ref[0] = jnp.dot(t, wc_ref[...], preferred_element_type=jnp.float32) + cbb_ref[...]
    dl_ref[0] = jnp.dot(t, wb_ref[...], preferred_element_type=jnp.float32) + bbb_ref[...]


def _nms_kernel(x1v_ref, y1v_ref, x2v_ref, y2v_ref,
                x1s_ref, y1s_ref, x2s_ref, y2s_ref, ord_ref,
                act_ref, supp_ref):
    n = pl.program_id(0)
    x1 = x1v_ref[0]
    y1 = y1v_ref[0]
    x2 = x2v_ref[0]
    y2 = y2v_ref[0]
    ar = _df_mul(_two_diff(x2, x1), _two_diff(y2, y1))
    ar0, ar1 = ar
    th = (jnp.float32(_TH_HI), jnp.float32(_TH_LO))
    eps = (jnp.float32(_EPS_HI), jnp.float32(_EPS_LO))
    idx = (lax.broadcasted_iota(jnp.int32, (8, 128), 0) * 128
           + lax.broadcasted_iota(jnp.int32, (8, 128), 1))
    supp_ref[...] = jnp.zeros((8, 128), jnp.float32)
    act_ref[0] = jnp.zeros((8, 128), jnp.float32)

    def body(t, c):
        i = ord_ref[n, t]
        xi1 = x1s_ref[n, i]
        yi1 = y1s_ref[n, i]
        xi2 = x2s_ref[n, i]
        yi2 = y2s_ref[n, i]
        supp = supp_ref[...]
        onehot = idx == i
        s_i = jnp.sum(jnp.where(onehot, supp, 0.0))
        active = s_i == 0.0
        xx1 = jnp.maximum(xi1, x1)
        yy1 = jnp.maximum(yi1, y1)
        xx2 = jnp.minimum(xi2, x2)
        yy2 = jnp.minimum(yi2, y2)
        w = _two_diff(xx2, xx1)
        h = _two_diff(yy2, yy1)
        w = (jnp.where(w[0] > 0, w[0], 0.0), jnp.where(w[0] > 0, w[1], 0.0))
        h = (jnp.where(h[0] > 0, h[0], 0.0), jnp.where(h[0] > 0, h[1], 0.0))
        inter = _df_mul(w, h)
        a0 = jnp.sum(jnp.where(onehot, ar0, 0.0))
        a1 = jnp.sum(jnp.where(onehot, ar1, 0.0))
        denom = _df_add(_df_add(_df_add((a0, a1), ar), _df_neg(inter)), eps)
        over = _df_gt(inter, _df_mul(denom, th))
        overf = jnp.where(over, 1.0, 0.0).astype(jnp.float32)
        supp_ref[...] = jnp.where(active, jnp.maximum(supp, overf), supp)
        af = jnp.where(active, jnp.float32(1.0), jnp.float32(0.0))
        act_ref[0] = jnp.where(idx == t, af, act_ref[0])
        return c

    lax.fori_loop(0, _PRE, body, 0)


def kernel(images, features, conv_w, conv_b, cls_w, cls_b, bbox_w, bbox_b):
    N = features.shape[0]
    img_h = float(images.shape[-2])
    img_w = float(images.shape[-1])

    xp = jnp.pad(features, ((0, 0), (0, 0), (1, 1), (1, 1)))
    x = xp.transpose(0, 2, 3, 1).reshape(N, _ROWS, 256)
    x = jnp.pad(x, ((0, 0), (53, _RPAD - _ROWS - 53), (0, 0)))
    wm = conv_w.transpose(2, 3, 1, 0).reshape(9, 256, 256)
    cb = conv_b.reshape(1, 256)
    wc = jnp.zeros((256, 128), jnp.float32).at[:, :_A].set(cls_w[:, :, 0, 0].T)
    cbb = jnp.zeros((1, 128), jnp.float32).at[0, :_A].set(cls_b)
    wb = jnp.zeros((256, 128), jnp.float32).at[:, :4 * _A].set(bbox_w[:, :, 0, 0].T)
    bbb = jnp.zeros((1, 128), jnp.float32).at[0, :4 * _A].set(bbox_b)

    obj_full, dl_full = pl.pallas_call(
        _head_kernel,
        grid=(N,),
        in_specs=[
            pl.BlockSpec((1, _RPAD, 256), lambda n: (n, 0, 0)),
            pl.BlockSpec((9, 256, 256), lambda n: (0, 0, 0)),
            pl.BlockSpec((1, 256), lambda n: (0, 0)),
            pl.BlockSpec((256, 128), lambda n: (0, 0)),
            pl.BlockSpec((1, 128), lambda n: (0, 0)),
            pl.BlockSpec((256, 128), lambda n: (0, 0)),
            pl.BlockSpec((1, 128), lambda n: (0, 0)),
        ],
        out_specs=[
            pl.BlockSpec((1, _ROWS, 128), lambda n: (n, 0, 0)),
            pl.BlockSpec((1, _ROWS, 128), lambda n: (n, 0, 0)),
        ],
        out_shape=[
            jax.ShapeDtypeStruct((N, _ROWS, 128), jnp.float32),
            jax.ShapeDtypeStruct((N, _ROWS, 128), jnp.float32),
        ],
    )(x, wm, cb, wc, cbb, wb, bbb)

    obj = obj_full.reshape(N, _HW, _HW, 128)[:, 1:51, 1:51, :_A].reshape(N, -1)
    dl = dl_full.reshape(N, _HW, _HW, 128)[:, 1:51, 1:51, :4 * _A].reshape(N, -1, 4)

    anchors = _mk_anchors()
    props = _decode_batched(dl, anchors)
    topv, topi = lax.top_k(obj, _PRE)
    boxes = jnp.take_along_axis(props, topi[..., None], axis=1)
    bx1 = jnp.clip(boxes[..., 0], 0.0, img_w)
    by1 = jnp.clip(boxes[..., 1], 0.0, img_h)
    bx2 = jnp.clip(boxes[..., 2], 0.0, img_w)
    by2 = jnp.clip(boxes[..., 3], 0.0, img_h)
    boxes = jnp.stack([bx1, by1, bx2, by2], axis=-1)
    ws = boxes[..., 2] - boxes[..., 0]
    hs = boxes[..., 3] - boxes[..., 1]
    valid = (ws >= _MIN_SIZE) & (hs >= _MIN_SIZE)
    order = jnp.argsort(jnp.logical_not(valid).astype(jnp.int32),
                        axis=1, stable=True).astype(jnp.int32)

    def pad_vec(a):
        return jnp.pad(a, ((0, 0), (0, 1024 - _PRE)))

    x1p = pad_vec(boxes[..., 0])
    y1p = pad_vec(boxes[..., 1])
    x2p = pad_vec(boxes[..., 2])
    y2p = pad_vec(boxes[..., 3])
    ordp = pad_vec(order)

    act = pl.pallas_call(
        _nms_kernel,
        grid=(N,),
        in_specs=[
            pl.BlockSpec((1, 8, 128), lambda n: (n, 0, 0)),
            pl.BlockSpec((1, 8, 128), lambda n: (n, 0, 0)),
            pl.BlockSpec((1, 8, 128), lambda n: (n, 0, 0)),
            pl.BlockSpec((1, 8, 128), lambda n: (n, 0, 0)),
            pl.BlockSpec(memory_space=pltpu.SMEM),
            pl.BlockSpec(memory_space=pltpu.SMEM),
            pl.BlockSpec(memory_space=pltpu.SMEM),
            pl.BlockSpec(memory_space=pltpu.SMEM),
            pl.BlockSpec(memory_space=pltpu.SMEM),
        ],
        out_specs=pl.BlockSpec((1, 8, 128), lambda n: (n, 0, 0)),
        out_shape=jax.ShapeDtypeStruct((N, 8, 128), jnp.float32),
        scratch_shapes=[pltpu.VMEM((8, 128), jnp.float32)],
    )(x1p.reshape(N, 8, 128), y1p.reshape(N, 8, 128),
      x2p.reshape(N, 8, 128), y2p.reshape(N, 8, 128),
      x1p, y1p, x2p, y2p, ordp)

    active = act.reshape(N, 1024)[:, :_PRE] > 0.5
    perm = jnp.argsort(jnp.logical_not(active).astype(jnp.int32),
                       axis=1, stable=True)
    keep = jnp.take_along_axis(order, perm, axis=1)
    count = jnp.sum(active.astype(jnp.int32), axis=1)
    pos = jnp.arange(_POST, dtype=jnp.int32)
    mask = pos[None, :] < count[:, None]
    sel = jnp.where(mask, keep, 0)
    kb = jnp.where(mask[..., None],
                   jnp.take_along_axis(boxes, sel[..., None], axis=1),
                   jnp.float32(0.0))
    return kb
